# baseline (speedup 1.0000x reference)
import functools
import jax
import jax.numpy as jnp
from jax import lax
from jax.experimental import pallas as pl
from jax.experimental.pallas import tpu as pltpu, tpu_sc as plsc

N_GRAPHS = 512
BASE_DIM = 128
CHARGE_PAD = 128
TBL = CHARGE_PAD + N_GRAPHS
NB = 4000

NW = 32
PER_W = 3200
CH = 128
NPAD = NW * PER_W


def _tables_kernel(ga_ref, ct_ref, cW1_ref, cb1_ref, cW2_ref, cb2_ref,
                   pW1_ref, pb1_ref, T_ref):
    f32 = jnp.float32
    pW1_c = pW1_ref[BASE_DIM:BASE_DIM + 16]
    pW1_h = pW1_ref[BASE_DIM + 16:]
    T_ref[:CHARGE_PAD] = jnp.dot(ct_ref[...], pW1_c, preferred_element_type=f32) + pb1_ref[...]
    h = jnp.dot(ga_ref[...], cW1_ref[...], preferred_element_type=f32) + cb1_ref[...]
    h = h * jax.nn.sigmoid(h)
    h = jnp.dot(h, cW2_ref[...], preferred_element_type=f32) + cb2_ref[...]
    T_ref[CHARGE_PAD:] = jnp.dot(h, pW1_h, preferred_element_type=f32)


def _sc_gather(T_hbm, bt_hbm, out_hbm, idxc, rows, sem):
    wid = lax.axis_index("s") * 2 + lax.axis_index("c")
    base = wid * PER_W

    def body(k, carry):
        off = base + k * CH
        pltpu.sync_copy(bt_hbm.at[pl.ds(off, CH)], idxc)
        pltpu.async_copy(T_hbm.at[idxc], rows, sem).wait()
        pltpu.sync_copy(rows, out_hbm.at[pl.ds(off, CH)])
        return carry

    lax.fori_loop(0, PER_W // CH, body, 0)


def _fused_kernel(sp_ref, ch_ref, b_ref, T_ref, pW1_ref, pW2_ref, pb2_ref, out_ref):
    f32 = jnp.float32
    dn = (((0,), (0,)), ((), ()))
    z = jnp.dot(sp_ref[...], pW1_ref[:BASE_DIM], preferred_element_type=f32)
    oh_c = (ch_ref[0] == jax.lax.broadcasted_iota(jnp.int32, (CHARGE_PAD, NB), 0)).astype(f32)
    z = z + jax.lax.dot_general(oh_c, T_ref[:CHARGE_PAD], dn, preferred_element_type=f32)
    z = z + b_ref[...]
    a = z * jax.nn.sigmoid(z)
    out_ref[...] = jnp.dot(a, pW2_ref[...], preferred_element_type=f32) + pb2_ref[...]


def kernel(species_emb, batch, charge, graph_attr, charge_table,
           cW1, cb1, cW2, cb2, pW1, pb1, pW2, pb2):
    n = species_emb.shape[0]
    grid = n // NB
    ch3d = charge.astype(jnp.int32).reshape(grid, 1, NB)
    bt_pad = jnp.zeros((NPAD,), jnp.int32).at[:n].set(batch.astype(jnp.int32) + CHARGE_PAD)
    ct_pad = jnp.zeros((CHARGE_PAD, charge_table.shape[1]), jnp.float32).at[:charge_table.shape[0]].set(charge_table)

    T = pl.pallas_call(
        _tables_kernel,
        out_shape=jax.ShapeDtypeStruct((TBL, BASE_DIM), jnp.float32),
    )(graph_attr, ct_pad, cW1, cb1.reshape(1, -1), cW2, cb2.reshape(1, -1),
      pW1, pb1.reshape(1, -1))

    mesh = plsc.VectorSubcoreMesh(core_axis_name="c", subcore_axis_name="s")
    B = pl.kernel(
        _sc_gather,
        out_type=jax.ShapeDtypeStruct((NPAD, BASE_DIM), jnp.float32),
        mesh=mesh,
        scratch_types=[
            pltpu.VMEM((CH,), jnp.int32),
            pltpu.VMEM((CH, BASE_DIM), jnp.float32),
            pltpu.SemaphoreType.DMA,
        ],
    )(T, bt_pad)

    full = lambda s: pl.BlockSpec(s, lambda i: (0, 0))
    out = pl.pallas_call(
        _fused_kernel,
        grid=(grid,),
        in_specs=[
            pl.BlockSpec((NB, BASE_DIM), lambda i: (i, 0)),
            pl.BlockSpec((1, 1, NB), lambda i: (i, 0, 0)),
            pl.BlockSpec((NB, BASE_DIM), lambda i: (i, 0)),
            full(T.shape),
            full(pW1.shape),
            full(pW2.shape),
            full((1, pb2.shape[0])),
        ],
        out_specs=pl.BlockSpec((NB, pW2.shape[1]), lambda i: (i, 0)),
        out_shape=jax.ShapeDtypeStruct((n, pW2.shape[1]), jnp.float32),
    )(species_emb, ch3d, B[:n], T, pW1, pW2, pb2.reshape(1, -1))
    return out


# speedup vs baseline: 1.1683x; 1.1683x over previous
import functools
import jax
import jax.numpy as jnp
from jax import lax
from jax.experimental import pallas as pl
from jax.experimental.pallas import tpu as pltpu, tpu_sc as plsc

N_GRAPHS = 512
BASE_DIM = 128
CHARGE_PAD = 128
TBL = CHARGE_PAD + N_GRAPHS
NB = 4000

NW = 32
PER_W = 3200
CH = 128
NPAD = NW * PER_W


def _tables_kernel(ga_ref, ct_ref, cW1_ref, cb1_ref, cW2_ref, cb2_ref,
                   pW1_ref, pb1_ref, T_ref):
    f32 = jnp.float32
    pW1_c = pW1_ref[BASE_DIM:BASE_DIM + 16]
    pW1_h = pW1_ref[BASE_DIM + 16:]
    T_ref[:CHARGE_PAD] = jnp.dot(ct_ref[...], pW1_c, preferred_element_type=f32) + pb1_ref[...]
    h = jnp.dot(ga_ref[...], cW1_ref[...], preferred_element_type=f32) + cb1_ref[...]
    h = h * jax.nn.sigmoid(h)
    h = jnp.dot(h, cW2_ref[...], preferred_element_type=f32) + cb2_ref[...]
    T_ref[CHARGE_PAD:] = jnp.dot(h, pW1_h, preferred_element_type=f32)


def _sc_gather(T_hbm, bt_hbm, out_hbm, idxa, idxb, rowsa, rowsb, sema, semb):
    wid = lax.axis_index("s") * 2 + lax.axis_index("c")
    base = wid * PER_W

    def load_start(k, idxc, rows, sem):
        off = base + k * CH
        pltpu.sync_copy(bt_hbm.at[pl.ds(off, CH)], idxc)
        return pltpu.async_copy(T_hbm.at[idxc], rows, sem)

    def store(k, rows):
        pltpu.sync_copy(rows, out_hbm.at[pl.ds(base + k * CH, CH)])

    load_start(0, idxa, rowsa, sema)

    def body(t, carry):
        load_start(2 * t + 1, idxb, rowsb, semb)
        pltpu.make_async_copy(T_hbm.at[idxa], rowsa, sema).wait()
        store(2 * t, rowsa)
        load_start(2 * t + 2, idxa, rowsa, sema)
        pltpu.make_async_copy(T_hbm.at[idxb], rowsb, semb).wait()
        store(2 * t + 1, rowsb)
        return carry

    lax.fori_loop(0, (PER_W // CH) // 2, body, 0)
    pltpu.make_async_copy(T_hbm.at[idxa], rowsa, sema).wait()
    store(PER_W // CH - 1, rowsa)


def _fused_kernel(sp_ref, ch_ref, b_ref, T_ref, pW1_ref, pW2_ref, pb2_ref, out_ref):
    f32 = jnp.float32
    dn = (((0,), (0,)), ((), ()))
    z = jnp.dot(sp_ref[...], pW1_ref[:BASE_DIM], preferred_element_type=f32)
    oh_c = (ch_ref[0] == jax.lax.broadcasted_iota(jnp.int32, (CHARGE_PAD, NB), 0)).astype(f32)
    z = z + jax.lax.dot_general(oh_c, T_ref[:CHARGE_PAD], dn, preferred_element_type=f32)
    z = z + b_ref[...]
    a = z * jax.nn.sigmoid(z)
    out_ref[...] = jnp.dot(a, pW2_ref[...], preferred_element_type=f32) + pb2_ref[...]


def kernel(species_emb, batch, charge, graph_attr, charge_table,
           cW1, cb1, cW2, cb2, pW1, pb1, pW2, pb2):
    n = species_emb.shape[0]
    grid = n // NB
    ch3d = charge.astype(jnp.int32).reshape(grid, 1, NB)
    bt_pad = jnp.zeros((NPAD,), jnp.int32).at[:n].set(batch.astype(jnp.int32) + CHARGE_PAD)
    ct_pad = jnp.zeros((CHARGE_PAD, charge_table.shape[1]), jnp.float32).at[:charge_table.shape[0]].set(charge_table)

    T = pl.pallas_call(
        _tables_kernel,
        out_shape=jax.ShapeDtypeStruct((TBL, BASE_DIM), jnp.float32),
    )(graph_attr, ct_pad, cW1, cb1.reshape(1, -1), cW2, cb2.reshape(1, -1),
      pW1, pb1.reshape(1, -1))

    mesh = plsc.VectorSubcoreMesh(core_axis_name="c", subcore_axis_name="s")
    B = pl.kernel(
        _sc_gather,
        out_type=jax.ShapeDtypeStruct((NPAD, BASE_DIM), jnp.float32),
        mesh=mesh,
        scratch_types=[
            pltpu.VMEM((CH,), jnp.int32),
            pltpu.VMEM((CH,), jnp.int32),
            pltpu.VMEM((CH, BASE_DIM), jnp.float32),
            pltpu.VMEM((CH, BASE_DIM), jnp.float32),
            pltpu.SemaphoreType.DMA,
            pltpu.SemaphoreType.DMA,
        ],
    )(T, bt_pad)

    full = lambda s: pl.BlockSpec(s, lambda i: (0, 0))
    out = pl.pallas_call(
        _fused_kernel,
        grid=(grid,),
        in_specs=[
            pl.BlockSpec((NB, BASE_DIM), lambda i: (i, 0)),
            pl.BlockSpec((1, 1, NB), lambda i: (i, 0, 0)),
            pl.BlockSpec((NB, BASE_DIM), lambda i: (i, 0)),
            full(T.shape),
            full(pW1.shape),
            full(pW2.shape),
            full((1, pb2.shape[0])),
        ],
        out_specs=pl.BlockSpec((NB, pW2.shape[1]), lambda i: (i, 0)),
        out_shape=jax.ShapeDtypeStruct((n, pW2.shape[1]), jnp.float32),
    )(species_emb, ch3d, B[:n], T, pW1, pW2, pb2.reshape(1, -1))
    return out


# speedup vs baseline: 5.1887x; 4.4412x over previous
import jax
import jax.numpy as jnp
from jax.experimental import pallas as pl

N_GRAPHS = 512
BASE_DIM = 128
CHARGE_PAD = 128
TBL = CHARGE_PAD + N_GRAPHS
NB = 4000


def _tables_kernel(ga_ref, ct_ref, cW1_ref, cb1_ref, cW2_ref, cb2_ref,
                   pW1_ref, pb1_ref, pW2_ref2, T_ref, W1s_ref, W2_ref):
    f32 = jnp.float32
    pW1_c = pW1_ref[BASE_DIM:BASE_DIM + 16]
    pW1_h = pW1_ref[BASE_DIM + 16:]
    T_ref[:CHARGE_PAD] = (jnp.dot(ct_ref[...], pW1_c, preferred_element_type=f32)
                          + pb1_ref[...]).astype(jnp.bfloat16)
    h = jnp.dot(ga_ref[...], cW1_ref[...], preferred_element_type=f32) + cb1_ref[...]
    h = h * jax.nn.sigmoid(h)
    h = jnp.dot(h, cW2_ref[...], preferred_element_type=f32) + cb2_ref[...]
    T_ref[CHARGE_PAD:] = jnp.dot(h, pW1_h, preferred_element_type=f32).astype(jnp.bfloat16)
    W1s_ref[...] = pW1_ref[:BASE_DIM].astype(jnp.bfloat16)
    W2_ref[...] = pW2_ref2[...].astype(jnp.bfloat16)


def _fused_kernel(sp_ref, ch_ref, bt_ref, T_ref, W1s_ref, W2_ref, pb2_ref, out_ref):
    f32 = jnp.float32
    bf16 = jnp.bfloat16
    dn = (((0,), (0,)), ((), ()))
    z = jnp.dot(sp_ref[...].astype(bf16), W1s_ref[...], preferred_element_type=f32)
    oh_c = (ch_ref[0] == jax.lax.broadcasted_iota(jnp.int32, (CHARGE_PAD, NB), 0)).astype(bf16)
    z = z + jax.lax.dot_general(oh_c, T_ref[:CHARGE_PAD], dn, preferred_element_type=f32)
    oh_b = (bt_ref[0] == jax.lax.broadcasted_iota(jnp.int32, (N_GRAPHS, NB), 0)).astype(bf16)
    z = z + jax.lax.dot_general(oh_b, T_ref[CHARGE_PAD:], dn, preferred_element_type=f32)
    a = z * jax.nn.sigmoid(z)
    out_ref[...] = jnp.dot(a.astype(bf16), W2_ref[...], preferred_element_type=f32) + pb2_ref[...]


def kernel(species_emb, batch, charge, graph_attr, charge_table,
           cW1, cb1, cW2, cb2, pW1, pb1, pW2, pb2):
    n = species_emb.shape[0]
    grid = n // NB
    ch3d = charge.astype(jnp.int32).reshape(grid, 1, NB)
    bt3d = batch.astype(jnp.int32).reshape(grid, 1, NB)
    ct_pad = jnp.zeros((CHARGE_PAD, charge_table.shape[1]), jnp.float32).at[:charge_table.shape[0]].set(charge_table)

    T, W1s, W2 = pl.pallas_call(
        _tables_kernel,
        out_shape=(jax.ShapeDtypeStruct((TBL, BASE_DIM), jnp.bfloat16),
                   jax.ShapeDtypeStruct((BASE_DIM, BASE_DIM), jnp.bfloat16),
                   jax.ShapeDtypeStruct((BASE_DIM, BASE_DIM), jnp.bfloat16)),
    )(graph_attr, ct_pad, cW1, cb1.reshape(1, -1), cW2, cb2.reshape(1, -1),
      pW1, pb1.reshape(1, -1), pW2)

    full = lambda s: pl.BlockSpec(s, lambda i: (0, 0))
    out = pl.pallas_call(
        _fused_kernel,
        grid=(grid,),
        in_specs=[
            pl.BlockSpec((NB, BASE_DIM), lambda i: (i, 0)),
            pl.BlockSpec((1, 1, NB), lambda i: (i, 0, 0)),
            pl.BlockSpec((1, 1, NB), lambda i: (i, 0, 0)),
            full(T.shape),
            full(W1s.shape),
            full(W2.shape),
            full((1, pb2.shape[0])),
        ],
        out_specs=pl.BlockSpec((NB, pW2.shape[1]), lambda i: (i, 0)),
        out_shape=jax.ShapeDtypeStruct((n, pW2.shape[1]), jnp.float32),
    )(species_emb, ch3d, bt3d, T, W1s, W2, pb2.reshape(1, -1))
    return out
